# Initial kernel scaffold; baseline (speedup 1.0000x reference)
import jax
import jax.numpy as jnp
from jax.experimental import pallas as pl


def kernel(Z, edge_index, rbf, emb_table, W_rbf, b_rbf, W_dense, b_dense):
    raise NotImplementedError("write your pallas kernel here")


# speedup vs baseline: 4.4518x; 4.4518x over previous
import jax
import jax.numpy as jnp
from jax.experimental import pallas as pl
from jax.experimental.pallas import tpu as pltpu
from jax.experimental.pallas import tpu_sc as plsc


def _tables_body(z_ref, emb_ref, wt1_ref, wt2_ref, wt3_ref, wrbft_ref,
                 brbf_ref, bdense_ref, h_ref, tab_ref, c_ref, bias_ref):
    n = z_ref.shape[0]
    v = emb_ref.shape[0]
    emb = emb_ref[...]
    if v < 128:
        emb = jnp.concatenate(
            [emb, jnp.zeros((128 - v, emb.shape[1]), emb.dtype)], axis=0)
    lane = jax.lax.broadcasted_iota(jnp.int32, (n, 128), 1)
    oh = (z_ref[...] == lane).astype(jnp.float32)
    h = jnp.dot(oh, emb, preferred_element_type=jnp.float32)
    h_ref[...] = h
    tab_ref[:n, :] = jnp.dot(h, wt1_ref[...],
                             preferred_element_type=jnp.float32)
    tab_ref[n:, :] = jnp.dot(h, wt2_ref[...],
                             preferred_element_type=jnp.float32)
    c_ref[...] = jnp.dot(wrbft_ref[...], wt3_ref[...],
                         preferred_element_type=jnp.float32)
    bias_ref[...] = jnp.dot(brbf_ref[...], wt3_ref[...],
                            preferred_element_type=jnp.float32) + bdense_ref[...]


def _edge_body(ga_ref, gb_ref, rbf_ref, c_ref, bias_ref, m_ref):
    acc = jnp.dot(rbf_ref[...], c_ref[...],
                  preferred_element_type=jnp.float32)
    m_ref[...] = acc + ga_ref[...] + gb_ref[...] + bias_ref[...]


def kernel(Z, edge_index, rbf, emb_table, W_rbf, b_rbf, W_dense, b_dense):
    N = Z.shape[0]
    E = edge_index.shape[1]
    EMB = emb_table.shape[1]
    NR = rbf.shape[1]

    Wd_t = W_dense.T
    Wt1 = Wd_t[:EMB]
    Wt2 = Wd_t[EMB:2 * EMB]
    Wt3 = Wd_t[2 * EMB:]
    Wrbf_t = W_rbf.T
    Zc = Z.reshape(N, 1).astype(jnp.int32)

    h, tab, C, bias = pl.pallas_call(
        _tables_body,
        out_shape=(
            jax.ShapeDtypeStruct((N, EMB), jnp.float32),
            jax.ShapeDtypeStruct((2 * N, EMB), jnp.float32),
            jax.ShapeDtypeStruct((NR, EMB), jnp.float32),
            jax.ShapeDtypeStruct((1, EMB), jnp.float32),
        ),
    )(Zc, emb_table, Wt1, Wt2, Wt3, Wrbf_t,
      b_rbf.reshape(1, EMB), b_dense.reshape(1, EMB))

    idx = jnp.concatenate([edge_index[0], edge_index[1] + N]).astype(jnp.int32)
    idx2d = idx.reshape(1, 2 * E)

    mesh = plsc.VectorSubcoreMesh(core_axis_name="c", subcore_axis_name="s")
    W = 128

    @pl.kernel(out_type=jax.ShapeDtypeStruct((2 * E, EMB), jnp.float32),
               mesh=mesh)
    def _gather(tab_hbm, i_hbm, o_hbm):
        def body(i_vmem, o_vmem):
            pltpu.sync_copy(tab_hbm.at[i_vmem.at[0]], o_vmem)

        pltpu.emit_pipeline(
            body,
            grid=(2 * E // W,),
            in_specs=[pl.BlockSpec((1, W), lambda i: (0, i))],
            out_specs=[pl.BlockSpec((W, EMB), lambda i: (i, 0))],
            core_axis_name=("c", "s"),
            dimension_semantics=(pltpu.PARALLEL,),
        )(i_hbm, o_hbm)

    g = _gather(tab, idx2d)

    BE = 2560
    NB = E // BE
    m = pl.pallas_call(
        _edge_body,
        grid=(NB,),
        in_specs=[
            pl.BlockSpec((BE, EMB), lambda i: (i, 0)),
            pl.BlockSpec((BE, EMB), lambda i: (i + NB, 0)),
            pl.BlockSpec((BE, NR), lambda i: (i, 0)),
            pl.BlockSpec((NR, EMB), lambda i: (0, 0)),
            pl.BlockSpec((1, EMB), lambda i: (0, 0)),
        ],
        out_specs=pl.BlockSpec((BE, EMB), lambda i: (i, 0)),
        out_shape=jax.ShapeDtypeStruct((E, EMB), jnp.float32),
    )(g, g, rbf, C, bias)

    return (h, m)
